# baseline (speedup 1.0000x reference)
import functools

import jax
import jax.numpy as jnp
from jax import lax
from jax.experimental import pallas as pl
from jax.experimental.pallas import tpu as pltpu
from jax.experimental.pallas import tpu_sc as plsc

_VOCAB = 1000000
_DIM = 64
_SCALE = 8.0
_LANES = 16


@functools.lru_cache(maxsize=None)
def _make_kernel(batch_flat: int):
    info = plsc.get_sparse_core_info()
    nc, ns = info.num_cores, info.num_subcores
    nw = nc * ns
    assert batch_flat % nw == 0
    b_per_w = batch_flat // nw
    chunk = 512
    assert b_per_w % chunk == 0
    n_chunks = b_per_w // chunk
    vecs_per_row = _DIM // _LANES

    mesh = plsc.VectorSubcoreMesh(core_axis_name="c", subcore_axis_name="s")

    @functools.partial(
        pl.kernel,
        mesh=mesh,
        compiler_params=pltpu.CompilerParams(use_tc_tiling_on_sc=False),
        out_type=jax.ShapeDtypeStruct((batch_flat, _DIM), jnp.float32),
        scratch_types=[
            pltpu.VMEM((chunk,), jnp.int32),
            pltpu.VMEM((chunk, _DIM), jnp.float32),
            pltpu.SemaphoreType.DMA,
        ],
    )
    def k(ids_hbm, table_hbm, out_hbm, idx_v, rows_v, sem):
        wid = lax.axis_index("s") * nc + lax.axis_index("c")
        base = wid * b_per_w

        def chunk_body(g, carry):
            off = base + g * chunk
            pltpu.sync_copy(ids_hbm.at[pl.ds(off, chunk)], idx_v)
            pltpu.async_copy(table_hbm.at[idx_v], rows_v, sem).wait()

            def row_body(r, c2):
                for v in range(vecs_per_row):
                    sl = pl.ds(v * _LANES, _LANES)
                    rows_v[r, sl] = rows_v[r, sl] * _SCALE
                return c2

            lax.fori_loop(0, chunk, row_body, 0)
            pltpu.sync_copy(rows_v, out_hbm.at[pl.ds(off, chunk)])
            return carry

        lax.fori_loop(0, n_chunks, chunk_body, 0)

    return k


def kernel(input_ids, table):
    b, h = input_ids.shape
    flat_ids = input_ids.reshape(b * h).astype(jnp.int32)
    out = _make_kernel(b * h)(flat_ids, table)
    return out.reshape(b, h, _DIM)


# speedup vs baseline: 1.1331x; 1.1331x over previous
import functools

import jax
import jax.numpy as jnp
from jax import lax
from jax.experimental import pallas as pl
from jax.experimental.pallas import tpu as pltpu
from jax.experimental.pallas import tpu_sc as plsc

_DIM = 64
_SCALE = 8.0
_LANES = 16
_CHUNK = 640


@functools.lru_cache(maxsize=None)
def _make_kernel(batch_flat: int):
    info = plsc.get_sparse_core_info()
    nc, ns = info.num_cores, info.num_subcores
    nw = nc * ns
    assert batch_flat % nw == 0
    b_per_w = batch_flat // nw
    chunk = _CHUNK
    assert b_per_w % (2 * chunk) == 0
    n_chunks = b_per_w // chunk
    vecs_per_row = _DIM // _LANES

    mesh = plsc.VectorSubcoreMesh(core_axis_name="c", subcore_axis_name="s")

    @functools.partial(
        pl.kernel,
        mesh=mesh,
        compiler_params=pltpu.CompilerParams(use_tc_tiling_on_sc=False),
        out_type=jax.ShapeDtypeStruct((batch_flat, _DIM), jnp.float32),
        scratch_types=[
            pltpu.VMEM((b_per_w,), jnp.int32),
            pltpu.VMEM((2, chunk, _DIM), jnp.float32),
            pltpu.SemaphoreType.DMA,
            pltpu.SemaphoreType.DMA,
        ],
    )
    def k(ids_hbm, table_hbm, out_hbm, idx_v, rows_v, gsem0, gsem1):
        wid = lax.axis_index("s") * nc + lax.axis_index("c")
        base = wid * b_per_w
        gsems = (gsem0, gsem1)

        pltpu.sync_copy(ids_hbm.at[pl.ds(base, b_per_w)], idx_v)
        pltpu.async_copy(
            table_hbm.at[idx_v.at[pl.ds(0, chunk)]], rows_v.at[0], gsem0
        )

        def pair_body(p, carry):
            for b in range(2):
                g = 2 * p + b
                nb = 1 - b

                @pl.when(g + 1 < n_chunks)
                def _start_next():
                    pltpu.async_copy(
                        table_hbm.at[idx_v.at[pl.ds((g + 1) * chunk, chunk)]],
                        rows_v.at[nb],
                        gsems[nb],
                    )

                pltpu.make_async_copy(
                    table_hbm.at[idx_v.at[pl.ds(0, chunk)]],
                    rows_v.at[b],
                    gsems[b],
                ).wait()

                buf = rows_v.at[b]

                @plsc.parallel_loop(0, chunk, unroll=4)
                def _scale_row(r):
                    for v in range(vecs_per_row):
                        sl = pl.ds(v * _LANES, _LANES)
                        buf[r, sl] = buf[r, sl] * _SCALE

                pltpu.sync_copy(
                    rows_v.at[b], out_hbm.at[pl.ds(base + g * chunk, chunk)]
                )
            return carry

        lax.fori_loop(0, n_chunks // 2, pair_body, 0)

    return k


def kernel(input_ids, table):
    b, h = input_ids.shape
    flat_ids = input_ids.reshape(b * h).astype(jnp.int32)
    out = _make_kernel(b * h)(flat_ids, table)
    return out.reshape(b, h, _DIM)
